# baseline (speedup 1.0000x reference)
import functools

import jax
import jax.numpy as jnp
from jax import lax
from jax.experimental import pallas as pl
from jax.experimental.pallas import tpu as pltpu
from jax.experimental.pallas import tpu_sc as plsc

_N = 10000
_NP = 10240
_E = 320000
_D = 128

_NC = 2
_NS = 16
_NW = _NC * _NS
_EPW = _E // _NW
_K = 40
_CHUNKS = _EPW // _K
_RPT = _NP // _NS


_G = 8
_GROUPS = _CHUNKS // _G


def _segsum_kernel(table, edges, zeros, out, acc, idxs, rows, sem_i, sem_g):
    c = lax.axis_index("c")
    s = lax.axis_index("s")
    pltpu.sync_copy(zeros, acc.at[pl.ds(s * _RPT, _RPT)])

    wid = c * _NS + s
    plsc.subcore_barrier()

    def body(j, carry):
        jj = j * _G
        di = [pltpu.async_copy(edges.at[wid, jj + g], idxs[g], sem_i[g])
              for g in range(_G)]
        dg = []
        for g in range(_G):
            di[g].wait()
            dg.append(pltpu.async_copy(table.at[idxs[g].at[0]], rows[g],
                                       sem_g[g]))
        for g in range(_G):
            dg[g].wait()
            pltpu.sync_copy(rows[g], acc.at[idxs[g].at[1]], add=True)
        return carry

    lax.fori_loop(0, _GROUPS, body, 0)
    for t in range(_GROUPS * _G, _CHUNKS):
        pltpu.async_copy(edges.at[wid, t], idxs[0], sem_i[0]).wait()
        pltpu.async_copy(table.at[idxs[0].at[0]], rows[0], sem_g[0]).wait()
        pltpu.sync_copy(rows[0], acc.at[idxs[0].at[1]], add=True)

    plsc.subcore_barrier()
    pltpu.sync_copy(acc.at[pl.ds(s * _RPT, _RPT)],
                    out.at[c, pl.ds(s * _RPT, _RPT)])


_segsum = functools.partial(
    pl.kernel,
    out_type=jax.ShapeDtypeStruct((_NC, _NP, _D), jnp.float32),
    mesh=plsc.VectorSubcoreMesh(core_axis_name="c", subcore_axis_name="s"),
    scratch_types=[
        pltpu.VMEM_SHARED((_NP, _D), jnp.float32),
        [pltpu.VMEM((2, _K), jnp.int32) for _ in range(_G)],
        [pltpu.VMEM((_K, _D), jnp.float32) for _ in range(_G)],
        [pltpu.SemaphoreType.DMA for _ in range(_G)],
        [pltpu.SemaphoreType.DMA for _ in range(_G)],
    ],
)(_segsum_kernel)


_R = 400


def _dense_body(p_ref, x_ref, wr_ref, wt_ref, b_ref, o_ref):
    agg = p_ref[0] + p_ref[1]
    acc = lax.dot_general(agg, wr_ref[...], (((1,), (1,)), ((), ())),
                          preferred_element_type=jnp.float32)
    acc = acc + lax.dot_general(x_ref[...], wt_ref[...],
                                (((1,), (1,)), ((), ())),
                                preferred_element_type=jnp.float32)
    o_ref[...] = jnp.maximum(acc + b_ref[...], 0.0)


def _dense_heads_body(p_ref, x_ref, wr_ref, wt_ref, b_ref, wh_ref, bh_ref,
                      o_ref):
    agg = p_ref[0] + p_ref[1]
    acc = lax.dot_general(agg, wr_ref[...], (((1,), (1,)), ((), ())),
                          preferred_element_type=jnp.float32)
    acc = acc + lax.dot_general(x_ref[...], wt_ref[...],
                                (((1,), (1,)), ((), ())),
                                preferred_element_type=jnp.float32)
    h = jnp.maximum(acc + b_ref[...], 0.0)
    o_ref[...] = lax.dot_general(h, wh_ref[...], (((1,), (1,)), ((), ())),
                                 preferred_element_type=jnp.float32) + bh_ref[...]


def _dense_layer(parts, xp, w_rel, w_root, b):
    return pl.pallas_call(
        _dense_body,
        grid=(_N // _R,),
        in_specs=[
            pl.BlockSpec((_NC, _R, _D), lambda i: (0, i, 0)),
            pl.BlockSpec((_R, _D), lambda i: (i, 0)),
            pl.BlockSpec((_D, _D), lambda i: (0, 0)),
            pl.BlockSpec((_D, _D), lambda i: (0, 0)),
            pl.BlockSpec((1, _D), lambda i: (0, 0)),
        ],
        out_specs=pl.BlockSpec((_R, _D), lambda i: (i, 0)),
        out_shape=jax.ShapeDtypeStruct((_N, _D), jnp.float32),
    )(parts, xp, w_rel, w_root, b)


def _dense_layer_heads(parts, xp, w_rel, w_root, b, w_heads, b_heads):
    return pl.pallas_call(
        _dense_heads_body,
        grid=(_N // _R,),
        in_specs=[
            pl.BlockSpec((_NC, _R, _D), lambda i: (0, i, 0)),
            pl.BlockSpec((_R, _D), lambda i: (i, 0)),
            pl.BlockSpec((_D, _D), lambda i: (0, 0)),
            pl.BlockSpec((_D, _D), lambda i: (0, 0)),
            pl.BlockSpec((1, _D), lambda i: (0, 0)),
            pl.BlockSpec((_D, _D), lambda i: (0, 0)),
            pl.BlockSpec((1, _D), lambda i: (0, 0)),
        ],
        out_specs=pl.BlockSpec((_R, _D), lambda i: (i, 0)),
        out_shape=jax.ShapeDtypeStruct((_N, _D), jnp.float32),
    )(parts, xp, w_rel, w_root, b, w_heads, b_heads)


def kernel(x, edge_index, W1_rel, b1, W1_root, W2_rel, b2, W2_root,
           Wa, ba, Wo, bo):
    src = edge_index[0].reshape(_NW, _CHUNKS, _K)
    dst = edge_index[1].reshape(_NW, _CHUNKS, _K)
    edges = jnp.stack([src, dst], axis=2)
    zeros = jnp.zeros((_RPT, _D), jnp.float32)

    w_heads = jnp.zeros((_D, _D), jnp.float32)
    w_heads = w_heads.at[:2, :].set(Wa).at[2, :].set(Wo[0])
    b_heads = jnp.zeros((_D,), jnp.float32)
    b_heads = b_heads.at[:2].set(ba).at[2].set(bo[0])

    parts1 = _segsum(x, edges, zeros)
    h1 = _dense_layer(parts1, x, W1_rel, W1_root, b1.reshape(1, _D))
    parts2 = _segsum(h1, edges, zeros)
    out = _dense_layer_heads(parts2, h1, W2_rel, W2_root, b2.reshape(1, _D),
                             w_heads, b_heads.reshape(1, _D))
    task_allocation = out[:, :2]
    task_order = out[:, 2:3]
    return (task_allocation, task_order)


# speedup vs baseline: 1.0307x; 1.0307x over previous
import functools

import jax
import jax.numpy as jnp
from jax import lax
from jax.experimental import pallas as pl
from jax.experimental.pallas import tpu as pltpu
from jax.experimental.pallas import tpu_sc as plsc

_N = 10000
_NP = 10240
_E = 320000
_D = 128

_NC = 2
_NS = 16
_NW = _NC * _NS
_EPW = _E // _NW
_K = 80
_CHUNKS = _EPW // _K
_RPT = _NP // _NS


_G = 4
_GROUPS = _CHUNKS // _G


def _segsum_kernel(table, edges, zeros, out, acc, idxs, rows, sem_i, sem_g):
    c = lax.axis_index("c")
    s = lax.axis_index("s")
    pltpu.sync_copy(zeros, acc.at[pl.ds(s * _RPT, _RPT)])

    wid = c * _NS + s
    plsc.subcore_barrier()

    def body(j, carry):
        jj = j * _G
        di = [pltpu.async_copy(edges.at[wid, jj + g], idxs[g], sem_i[g])
              for g in range(_G)]
        dg = []
        for g in range(_G):
            di[g].wait()
            dg.append(pltpu.async_copy(table.at[idxs[g].at[0]], rows[g],
                                       sem_g[g]))
        for g in range(_G):
            dg[g].wait()
            pltpu.sync_copy(rows[g], acc.at[idxs[g].at[1]], add=True)
        return carry

    lax.fori_loop(0, _GROUPS, body, 0)
    for t in range(_GROUPS * _G, _CHUNKS):
        pltpu.async_copy(edges.at[wid, t], idxs[0], sem_i[0]).wait()
        pltpu.async_copy(table.at[idxs[0].at[0]], rows[0], sem_g[0]).wait()
        pltpu.sync_copy(rows[0], acc.at[idxs[0].at[1]], add=True)

    plsc.subcore_barrier()
    pltpu.sync_copy(acc.at[pl.ds(s * _RPT, _RPT)],
                    out.at[c, pl.ds(s * _RPT, _RPT)])


_segsum = functools.partial(
    pl.kernel,
    out_type=jax.ShapeDtypeStruct((_NC, _NP, _D), jnp.float32),
    mesh=plsc.VectorSubcoreMesh(core_axis_name="c", subcore_axis_name="s"),
    scratch_types=[
        pltpu.VMEM_SHARED((_NP, _D), jnp.float32),
        [pltpu.VMEM((2, _K), jnp.int32) for _ in range(_G)],
        [pltpu.VMEM((_K, _D), jnp.float32) for _ in range(_G)],
        [pltpu.SemaphoreType.DMA for _ in range(_G)],
        [pltpu.SemaphoreType.DMA for _ in range(_G)],
    ],
)(_segsum_kernel)


_R = 400


def _root_body(x_ref, wt_ref, b_ref, o_ref):
    o_ref[...] = lax.dot_general(x_ref[...], wt_ref[...],
                                 (((1,), (1,)), ((), ())),
                                 preferred_element_type=jnp.float32) + b_ref[...]


def _combine_body(p_ref, r_ref, wr_ref, o_ref):
    agg = p_ref[0] + p_ref[1]
    acc = lax.dot_general(agg, wr_ref[...], (((1,), (1,)), ((), ())),
                          preferred_element_type=jnp.float32)
    o_ref[...] = jnp.maximum(acc + r_ref[...], 0.0)


def _combine_heads_body(p_ref, r_ref, wr_ref, wh_ref, bh_ref, o_ref):
    agg = p_ref[0] + p_ref[1]
    acc = lax.dot_general(agg, wr_ref[...], (((1,), (1,)), ((), ())),
                          preferred_element_type=jnp.float32)
    h = jnp.maximum(acc + r_ref[...], 0.0)
    o_ref[...] = lax.dot_general(h, wh_ref[...], (((1,), (1,)), ((), ())),
                                 preferred_element_type=jnp.float32) + bh_ref[...]


def _root_term(xp, w_root, b):
    return pl.pallas_call(
        _root_body,
        grid=(_N // _R,),
        in_specs=[
            pl.BlockSpec((_R, _D), lambda i: (i, 0)),
            pl.BlockSpec((_D, _D), lambda i: (0, 0)),
            pl.BlockSpec((1, _D), lambda i: (0, 0)),
        ],
        out_specs=pl.BlockSpec((_R, _D), lambda i: (i, 0)),
        out_shape=jax.ShapeDtypeStruct((_N, _D), jnp.float32),
    )(xp, w_root, b)


def _combine(parts, root, w_rel):
    return pl.pallas_call(
        _combine_body,
        grid=(_N // _R,),
        in_specs=[
            pl.BlockSpec((_NC, _R, _D), lambda i: (0, i, 0)),
            pl.BlockSpec((_R, _D), lambda i: (i, 0)),
            pl.BlockSpec((_D, _D), lambda i: (0, 0)),
        ],
        out_specs=pl.BlockSpec((_R, _D), lambda i: (i, 0)),
        out_shape=jax.ShapeDtypeStruct((_N, _D), jnp.float32),
    )(parts, root, w_rel)


def _combine_heads(parts, root, w_rel, w_heads, b_heads):
    return pl.pallas_call(
        _combine_heads_body,
        grid=(_N // _R,),
        in_specs=[
            pl.BlockSpec((_NC, _R, _D), lambda i: (0, i, 0)),
            pl.BlockSpec((_R, _D), lambda i: (i, 0)),
            pl.BlockSpec((_D, _D), lambda i: (0, 0)),
            pl.BlockSpec((_D, _D), lambda i: (0, 0)),
            pl.BlockSpec((1, _D), lambda i: (0, 0)),
        ],
        out_specs=pl.BlockSpec((_R, _D), lambda i: (i, 0)),
        out_shape=jax.ShapeDtypeStruct((_N, _D), jnp.float32),
    )(parts, root, w_rel, w_heads, b_heads)


def kernel(x, edge_index, W1_rel, b1, W1_root, W2_rel, b2, W2_root,
           Wa, ba, Wo, bo):
    src = edge_index[0].reshape(_NW, _CHUNKS, _K)
    dst = edge_index[1].reshape(_NW, _CHUNKS, _K)
    edges = jnp.stack([src, dst], axis=2)
    zeros = jnp.zeros((_RPT, _D), jnp.float32)

    w_heads = jnp.zeros((_D, _D), jnp.float32)
    w_heads = w_heads.at[:2, :].set(Wa).at[2, :].set(Wo[0])
    b_heads = jnp.zeros((_D,), jnp.float32)
    b_heads = b_heads.at[:2].set(ba).at[2].set(bo[0])

    parts1 = _segsum(x, edges, zeros)
    root1 = _root_term(x, W1_root, b1.reshape(1, _D))
    h1 = _combine(parts1, root1, W1_rel)
    parts2 = _segsum(h1, edges, zeros)
    root2 = _root_term(h1, W2_root, b2.reshape(1, _D))
    out = _combine_heads(parts2, root2, W2_rel, w_heads,
                         b_heads.reshape(1, _D))
    task_allocation = out[:, :2]
    task_order = out[:, 2:3]
    return (task_allocation, task_order)


# speedup vs baseline: 1.0932x; 1.0607x over previous
import functools

import jax
import jax.numpy as jnp
from jax import lax
from jax.experimental import pallas as pl
from jax.experimental.pallas import tpu as pltpu
from jax.experimental.pallas import tpu_sc as plsc

_N = 10000
_NP = 10240
_E = 320000
_D = 128

_NC = 2
_NS = 16
_NW = _NC * _NS
_EPW = _E // _NW
_K = 80
_CHUNKS = _EPW // _K
_RPT = _NP // _NS


_G = 4
_GL = 8
_GROUPS = _CHUNKS // _GL


def _segsum_kernel(table, src, dst, zeros, out, acc, srcg, dstg, rows,
                   sem_s, sem_d, sem_g):
    c = lax.axis_index("c")
    s = lax.axis_index("s")
    pltpu.sync_copy(zeros, acc.at[pl.ds(s * _RPT, _RPT)])

    wid = c * _NS + s
    plsc.subcore_barrier()

    def body(j, carry):
        jj = j * _GL
        ds_ = pltpu.async_copy(src.at[wid, pl.ds(jj, _GL)], srcg, sem_s)
        dd_ = pltpu.async_copy(dst.at[wid, pl.ds(jj, _GL)], dstg, sem_d)
        ds_.wait()
        dd_.wait()
        for h in range(_GL // _G):
            dg = [pltpu.async_copy(table.at[srcg.at[h * _G + g]], rows[g],
                                   sem_g[g]) for g in range(_G)]
            for g in range(_G):
                dg[g].wait()
                pltpu.sync_copy(rows[g], acc.at[dstg.at[h * _G + g]],
                                add=True)
        return carry

    lax.fori_loop(0, _GROUPS, body, 0)
    rem = _CHUNKS - _GROUPS * _GL
    if rem:
        t0 = _GROUPS * _GL
        pltpu.async_copy(src.at[wid, pl.ds(t0, rem)],
                         srcg.at[pl.ds(0, rem)], sem_s).wait()
        pltpu.async_copy(dst.at[wid, pl.ds(t0, rem)],
                         dstg.at[pl.ds(0, rem)], sem_d).wait()
        dg = [pltpu.async_copy(table.at[srcg.at[g]], rows[g % _G],
                               sem_g[g % _G]) for g in range(min(rem, _G))]
        for g in range(rem):
            dg[g].wait()
            pltpu.sync_copy(rows[g % _G], acc.at[dstg.at[g]], add=True)
            if g + _G < rem:
                dg.append(pltpu.async_copy(table.at[srcg.at[g + _G]],
                                           rows[g % _G], sem_g[g % _G]))

    plsc.subcore_barrier()
    pltpu.sync_copy(acc.at[pl.ds(s * _RPT, _RPT)],
                    out.at[c, pl.ds(s * _RPT, _RPT)])


_segsum = functools.partial(
    pl.kernel,
    out_type=jax.ShapeDtypeStruct((_NC, _NP, _D), jnp.float32),
    mesh=plsc.VectorSubcoreMesh(core_axis_name="c", subcore_axis_name="s"),
    scratch_types=[
        pltpu.VMEM_SHARED((_NP, _D), jnp.float32),
        pltpu.VMEM((_GL, _K), jnp.int32),
        pltpu.VMEM((_GL, _K), jnp.int32),
        [pltpu.VMEM((_K, _D), jnp.float32) for _ in range(_G)],
        pltpu.SemaphoreType.DMA,
        pltpu.SemaphoreType.DMA,
        [pltpu.SemaphoreType.DMA for _ in range(_G)],
    ],
)(_segsum_kernel)


_R = 400


def _root_body(x_ref, wt_ref, b_ref, o_ref):
    o_ref[...] = lax.dot_general(x_ref[...], wt_ref[...],
                                 (((1,), (1,)), ((), ())),
                                 preferred_element_type=jnp.float32) + b_ref[...]


def _combine_body(p_ref, r_ref, wr_ref, o_ref):
    agg = p_ref[0] + p_ref[1]
    acc = lax.dot_general(agg, wr_ref[...], (((1,), (1,)), ((), ())),
                          preferred_element_type=jnp.float32)
    o_ref[...] = jnp.maximum(acc + r_ref[...], 0.0)


def _combine_heads_body(p_ref, r_ref, wr_ref, wh_ref, bh_ref, o_ref):
    agg = p_ref[0] + p_ref[1]
    acc = lax.dot_general(agg, wr_ref[...], (((1,), (1,)), ((), ())),
                          preferred_element_type=jnp.float32)
    h = jnp.maximum(acc + r_ref[...], 0.0)
    o_ref[...] = lax.dot_general(h, wh_ref[...], (((1,), (1,)), ((), ())),
                                 preferred_element_type=jnp.float32) + bh_ref[...]


_HD = 8


def _root_term(xp, w_root, b):
    return pl.pallas_call(
        _root_body,
        grid=(_N // _R,),
        in_specs=[
            pl.BlockSpec((_R, _D), lambda i: (i, 0)),
            pl.BlockSpec((_D, _D), lambda i: (0, 0)),
            pl.BlockSpec((1, _D), lambda i: (0, 0)),
        ],
        out_specs=pl.BlockSpec((_R, _D), lambda i: (i, 0)),
        out_shape=jax.ShapeDtypeStruct((_N, _D), jnp.float32),
    )(xp, w_root, b)


def _combine(parts, root, w_rel):
    return pl.pallas_call(
        _combine_body,
        grid=(_N // _R,),
        in_specs=[
            pl.BlockSpec((_NC, _R, _D), lambda i: (0, i, 0)),
            pl.BlockSpec((_R, _D), lambda i: (i, 0)),
            pl.BlockSpec((_D, _D), lambda i: (0, 0)),
        ],
        out_specs=pl.BlockSpec((_R, _D), lambda i: (i, 0)),
        out_shape=jax.ShapeDtypeStruct((_N, _D), jnp.float32),
    )(parts, root, w_rel)


def _combine_heads(parts, root, w_rel, w_heads, b_heads):
    return pl.pallas_call(
        _combine_heads_body,
        grid=(_N // _R,),
        in_specs=[
            pl.BlockSpec((_NC, _R, _D), lambda i: (0, i, 0)),
            pl.BlockSpec((_R, _D), lambda i: (i, 0)),
            pl.BlockSpec((_D, _D), lambda i: (0, 0)),
            pl.BlockSpec((_HD, _D), lambda i: (0, 0)),
            pl.BlockSpec((1, _HD), lambda i: (0, 0)),
        ],
        out_specs=pl.BlockSpec((_R, _HD), lambda i: (i, 0)),
        out_shape=jax.ShapeDtypeStruct((_N, _HD), jnp.float32),
    )(parts, root, w_rel, w_heads, b_heads)


def kernel(x, edge_index, W1_rel, b1, W1_root, W2_rel, b2, W2_root,
           Wa, ba, Wo, bo):
    src = edge_index[0].reshape(_NW, _CHUNKS, _K)
    dst = edge_index[1].reshape(_NW, _CHUNKS, _K)
    zeros = jnp.zeros((_RPT, _D), jnp.float32)

    w_heads = jnp.zeros((_HD, _D), jnp.float32)
    w_heads = w_heads.at[:2, :].set(Wa).at[2, :].set(Wo[0])
    b_heads = jnp.zeros((_HD,), jnp.float32)
    b_heads = b_heads.at[:2].set(ba).at[2].set(bo[0])

    parts1 = _segsum(x, src, dst, zeros)
    root1 = _root_term(x, W1_root, b1.reshape(1, _D))
    h1 = _combine(parts1, root1, W1_rel)
    parts2 = _segsum(h1, src, dst, zeros)
    root2 = _root_term(h1, W2_root, b2.reshape(1, _D))
    out = _combine_heads(parts2, root2, W2_rel, w_heads,
                         b_heads.reshape(1, _HD))
    task_allocation = out[:, :2]
    task_order = out[:, 2:3]
    return (task_allocation, task_order)


# speedup vs baseline: 1.1461x; 1.0483x over previous
import functools

import jax
import jax.numpy as jnp
from jax import lax
from jax.experimental import pallas as pl
from jax.experimental.pallas import tpu as pltpu
from jax.experimental.pallas import tpu_sc as plsc

_N = 10000
_NP = 10240
_E = 320000
_D = 128

_NC = 2
_NS = 16
_NW = _NC * _NS
_EPW = _E // _NW
_K = 80
_CHUNKS = _EPW // _K
_RPT = _NP // _NS


_G = 4
_GL = 8
_GROUPS = _CHUNKS // _GL


def _segsum_kernel(table, src, dst, zeros, out, acc, srcg, dstg, rows,
                   sem_s, sem_d, sem_g):
    c = lax.axis_index("c")
    s = lax.axis_index("s")
    pltpu.sync_copy(zeros, acc.at[pl.ds(s * _RPT, _RPT)])

    wid = c * _NS + s
    plsc.subcore_barrier()

    def body(j, carry):
        jj = j * _GL
        ds_ = pltpu.async_copy(src.at[wid, pl.ds(jj, _GL)], srcg, sem_s)
        dd_ = pltpu.async_copy(dst.at[wid, pl.ds(jj, _GL)], dstg, sem_d)
        ds_.wait()
        dd_.wait()
        for h in range(_GL // _G):
            dg = [pltpu.async_copy(table.at[srcg.at[h * _G + g]], rows[g],
                                   sem_g[g]) for g in range(_G)]
            for g in range(_G):
                dg[g].wait()
                pltpu.sync_copy(rows[g], acc.at[dstg.at[h * _G + g]],
                                add=True)
        return carry

    lax.fori_loop(0, _GROUPS, body, 0)
    rem = _CHUNKS - _GROUPS * _GL
    if rem:
        t0 = _GROUPS * _GL
        pltpu.async_copy(src.at[wid, pl.ds(t0, rem)],
                         srcg.at[pl.ds(0, rem)], sem_s).wait()
        pltpu.async_copy(dst.at[wid, pl.ds(t0, rem)],
                         dstg.at[pl.ds(0, rem)], sem_d).wait()
        dg = [pltpu.async_copy(table.at[srcg.at[g]], rows[g % _G],
                               sem_g[g % _G]) for g in range(min(rem, _G))]
        for g in range(rem):
            dg[g].wait()
            pltpu.sync_copy(rows[g % _G], acc.at[dstg.at[g]], add=True)
            if g + _G < rem:
                dg.append(pltpu.async_copy(table.at[srcg.at[g + _G]],
                                           rows[g % _G], sem_g[g % _G]))

    plsc.subcore_barrier()
    pltpu.sync_copy(acc.at[pl.ds(s * _RPT, _RPT)],
                    out.at[c, pl.ds(s * _RPT, _RPT)])


_segsum = functools.partial(
    pl.kernel,
    out_type=jax.ShapeDtypeStruct((_NC, _NP, _D), jnp.float32),
    mesh=plsc.VectorSubcoreMesh(core_axis_name="c", subcore_axis_name="s"),
    scratch_types=[
        pltpu.VMEM_SHARED((_NP, _D), jnp.float32),
        pltpu.VMEM((_GL, _K), jnp.int32),
        pltpu.VMEM((_GL, _K), jnp.int32),
        [pltpu.VMEM((_K, _D), jnp.float32) for _ in range(_G)],
        pltpu.SemaphoreType.DMA,
        pltpu.SemaphoreType.DMA,
        [pltpu.SemaphoreType.DMA for _ in range(_G)],
    ],
)(_segsum_kernel)


_R = 2000


def _root_body(x_ref, wt_ref, b_ref, o_ref):
    o_ref[...] = lax.dot_general(x_ref[...], wt_ref[...],
                                 (((1,), (1,)), ((), ())),
                                 preferred_element_type=jnp.float32) + b_ref[...]


def _combine_body(p_ref, r_ref, wr_ref, o_ref):
    agg = p_ref[0] + p_ref[1]
    acc = lax.dot_general(agg, wr_ref[...], (((1,), (1,)), ((), ())),
                          preferred_element_type=jnp.float32)
    o_ref[...] = jnp.maximum(acc + r_ref[...], 0.0)


def _combine_heads_body(p_ref, r_ref, wr_ref, wh_ref, bh_ref, o_ref):
    agg = p_ref[0] + p_ref[1]
    acc = lax.dot_general(agg, wr_ref[...], (((1,), (1,)), ((), ())),
                          preferred_element_type=jnp.float32)
    h = jnp.maximum(acc + r_ref[...], 0.0)
    o_ref[...] = lax.dot_general(h, wh_ref[...], (((1,), (1,)), ((), ())),
                                 preferred_element_type=jnp.float32) + bh_ref[...]


_HD = 8


def _root_term(xp, w_root, b):
    return pl.pallas_call(
        _root_body,
        grid=(_N // _R,),
        in_specs=[
            pl.BlockSpec((_R, _D), lambda i: (i, 0)),
            pl.BlockSpec((_D, _D), lambda i: (0, 0)),
            pl.BlockSpec((1, _D), lambda i: (0, 0)),
        ],
        out_specs=pl.BlockSpec((_R, _D), lambda i: (i, 0)),
        out_shape=jax.ShapeDtypeStruct((_N, _D), jnp.float32),
    )(xp, w_root, b)


def _combine(parts, root, w_rel):
    return pl.pallas_call(
        _combine_body,
        grid=(_N // _R,),
        in_specs=[
            pl.BlockSpec((_NC, _R, _D), lambda i: (0, i, 0)),
            pl.BlockSpec((_R, _D), lambda i: (i, 0)),
            pl.BlockSpec((_D, _D), lambda i: (0, 0)),
        ],
        out_specs=pl.BlockSpec((_R, _D), lambda i: (i, 0)),
        out_shape=jax.ShapeDtypeStruct((_N, _D), jnp.float32),
    )(parts, root, w_rel)


def _combine_heads(parts, root, w_rel, w_heads, b_heads):
    return pl.pallas_call(
        _combine_heads_body,
        grid=(_N // _R,),
        in_specs=[
            pl.BlockSpec((_NC, _R, _D), lambda i: (0, i, 0)),
            pl.BlockSpec((_R, _D), lambda i: (i, 0)),
            pl.BlockSpec((_D, _D), lambda i: (0, 0)),
            pl.BlockSpec((_HD, _D), lambda i: (0, 0)),
            pl.BlockSpec((1, _HD), lambda i: (0, 0)),
        ],
        out_specs=pl.BlockSpec((_R, _HD), lambda i: (i, 0)),
        out_shape=jax.ShapeDtypeStruct((_N, _HD), jnp.float32),
    )(parts, root, w_rel, w_heads, b_heads)


def kernel(x, edge_index, W1_rel, b1, W1_root, W2_rel, b2, W2_root,
           Wa, ba, Wo, bo):
    src = edge_index[0].reshape(_NW, _CHUNKS, _K)
    dst = edge_index[1].reshape(_NW, _CHUNKS, _K)
    zeros = jnp.zeros((_RPT, _D), jnp.float32)

    w_heads = jnp.zeros((_HD, _D), jnp.float32)
    w_heads = w_heads.at[:2, :].set(Wa).at[2, :].set(Wo[0])
    b_heads = jnp.zeros((_HD,), jnp.float32)
    b_heads = b_heads.at[:2].set(ba).at[2].set(bo[0])

    parts1 = _segsum(x, src, dst, zeros)
    root1 = _root_term(x, W1_root, b1.reshape(1, _D))
    h1 = _combine(parts1, root1, W1_rel)
    parts2 = _segsum(h1, src, dst, zeros)
    root2 = _root_term(h1, W2_root, b2.reshape(1, _D))
    out = _combine_heads(parts2, root2, W2_rel, w_heads,
                         b_heads.reshape(1, _HD))
    task_allocation = out[:, :2]
    task_order = out[:, 2:3]
    return (task_allocation, task_order)


# speedup vs baseline: 1.1860x; 1.0348x over previous
import functools

import jax
import jax.numpy as jnp
from jax import lax
from jax.experimental import pallas as pl
from jax.experimental.pallas import tpu as pltpu
from jax.experimental.pallas import tpu_sc as plsc

_N = 10000
_NP = 10240
_E = 320000
_D = 128

_NC = 2
_NS = 16
_NW = _NC * _NS
_EPW = _E // _NW
_K = 80
_CHUNKS = _EPW // _K
_RPT = _NP // _NS


_G = 4
_GL = 8
_GROUPS = _CHUNKS // _GL


def _segsum_kernel(table, e4, zeros, out, acc, srcg, dstg, rows,
                   sem_s, sem_d, sem_g):
    c = lax.axis_index("c")
    s = lax.axis_index("s")
    pltpu.sync_copy(zeros, acc.at[pl.ds(s * _RPT, _RPT)])

    wid = c * _NS + s
    plsc.subcore_barrier()

    def body(j, carry):
        jj = j * _GL
        ds_ = pltpu.async_copy(e4.at[0, wid, pl.ds(jj, _GL)], srcg, sem_s)
        dd_ = pltpu.async_copy(e4.at[1, wid, pl.ds(jj, _GL)], dstg, sem_d)
        ds_.wait()
        dd_.wait()
        for h in range(_GL // _G):
            dg = [pltpu.async_copy(table.at[srcg.at[h * _G + g]], rows[g],
                                   sem_g[g]) for g in range(_G)]
            for g in range(_G):
                dg[g].wait()
                pltpu.sync_copy(rows[g], acc.at[dstg.at[h * _G + g]],
                                add=True)
        return carry

    lax.fori_loop(0, _GROUPS, body, 0)
    rem = _CHUNKS - _GROUPS * _GL
    if rem:
        t0 = _GROUPS * _GL
        pltpu.async_copy(e4.at[0, wid, pl.ds(t0, rem)],
                         srcg.at[pl.ds(0, rem)], sem_s).wait()
        pltpu.async_copy(e4.at[1, wid, pl.ds(t0, rem)],
                         dstg.at[pl.ds(0, rem)], sem_d).wait()
        dg = [pltpu.async_copy(table.at[srcg.at[g]], rows[g % _G],
                               sem_g[g % _G]) for g in range(min(rem, _G))]
        for g in range(rem):
            dg[g].wait()
            pltpu.sync_copy(rows[g % _G], acc.at[dstg.at[g]], add=True)
            if g + _G < rem:
                dg.append(pltpu.async_copy(table.at[srcg.at[g + _G]],
                                           rows[g % _G], sem_g[g % _G]))

    plsc.subcore_barrier()
    pltpu.sync_copy(acc.at[pl.ds(s * _RPT, _RPT)],
                    out.at[c, pl.ds(s * _RPT, _RPT)])


_segsum = functools.partial(
    pl.kernel,
    out_type=jax.ShapeDtypeStruct((_NC, _NP, _D), jnp.float32),
    mesh=plsc.VectorSubcoreMesh(core_axis_name="c", subcore_axis_name="s"),
    scratch_types=[
        pltpu.VMEM_SHARED((_NP, _D), jnp.float32),
        pltpu.VMEM((_GL, _K), jnp.int32),
        pltpu.VMEM((_GL, _K), jnp.int32),
        [pltpu.VMEM((_K, _D), jnp.float32) for _ in range(_G)],
        pltpu.SemaphoreType.DMA,
        pltpu.SemaphoreType.DMA,
        [pltpu.SemaphoreType.DMA for _ in range(_G)],
    ],
)(_segsum_kernel)


_R = 2000


def _root_body(x_ref, wt_ref, b_ref, o_ref):
    o_ref[...] = lax.dot_general(x_ref[...], wt_ref[...],
                                 (((1,), (1,)), ((), ())),
                                 preferred_element_type=jnp.float32) + b_ref[...]


def _combine_body(p_ref, r_ref, wr_ref, o_ref):
    agg = p_ref[0] + p_ref[1]
    acc = lax.dot_general(agg, wr_ref[...], (((1,), (1,)), ((), ())),
                          preferred_element_type=jnp.float32)
    o_ref[...] = jnp.maximum(acc + r_ref[...], 0.0)


def _combine_heads_body(p_ref, r_ref, wr_ref, wh_ref, bh_ref, oa_ref,
                        oo_ref):
    agg = p_ref[0] + p_ref[1]
    acc = lax.dot_general(agg, wr_ref[...], (((1,), (1,)), ((), ())),
                          preferred_element_type=jnp.float32)
    h = jnp.maximum(acc + r_ref[...], 0.0)
    heads = lax.dot_general(h, wh_ref[...], (((1,), (1,)), ((), ())),
                            preferred_element_type=jnp.float32) + bh_ref[...]
    oa_ref[...] = heads[:, :2]
    oo_ref[...] = heads[:, 2:3]


_HD = 8


def _root_term(xp, w_root, b):
    return pl.pallas_call(
        _root_body,
        grid=(_N // _R,),
        in_specs=[
            pl.BlockSpec((_R, _D), lambda i: (i, 0)),
            pl.BlockSpec((_D, _D), lambda i: (0, 0)),
            pl.BlockSpec((1, _D), lambda i: (0, 0)),
        ],
        out_specs=pl.BlockSpec((_R, _D), lambda i: (i, 0)),
        out_shape=jax.ShapeDtypeStruct((_N, _D), jnp.float32),
    )(xp, w_root, b)


def _combine(parts, root, w_rel):
    return pl.pallas_call(
        _combine_body,
        grid=(_N // _R,),
        in_specs=[
            pl.BlockSpec((_NC, _R, _D), lambda i: (0, i, 0)),
            pl.BlockSpec((_R, _D), lambda i: (i, 0)),
            pl.BlockSpec((_D, _D), lambda i: (0, 0)),
        ],
        out_specs=pl.BlockSpec((_R, _D), lambda i: (i, 0)),
        out_shape=jax.ShapeDtypeStruct((_N, _D), jnp.float32),
    )(parts, root, w_rel)


def _combine_heads(parts, root, w_rel, w_heads, b_heads):
    return pl.pallas_call(
        _combine_heads_body,
        grid=(_N // _R,),
        in_specs=[
            pl.BlockSpec((_NC, _R, _D), lambda i: (0, i, 0)),
            pl.BlockSpec((_R, _D), lambda i: (i, 0)),
            pl.BlockSpec((_D, _D), lambda i: (0, 0)),
            pl.BlockSpec((_HD, _D), lambda i: (0, 0)),
            pl.BlockSpec((1, _HD), lambda i: (0, 0)),
        ],
        out_specs=[pl.BlockSpec((_R, 2), lambda i: (i, 0)),
                   pl.BlockSpec((_R, 1), lambda i: (i, 0))],
        out_shape=[jax.ShapeDtypeStruct((_N, 2), jnp.float32),
                   jax.ShapeDtypeStruct((_N, 1), jnp.float32)],
    )(parts, root, w_rel, w_heads, b_heads)


def kernel(x, edge_index, W1_rel, b1, W1_root, W2_rel, b2, W2_root,
           Wa, ba, Wo, bo):
    e4 = edge_index.reshape(2, _NW, _CHUNKS, _K)
    zeros = jnp.zeros((_RPT, _D), jnp.float32)

    w_heads = jnp.zeros((_HD, _D), jnp.float32)
    w_heads = w_heads.at[:2, :].set(Wa).at[2, :].set(Wo[0])
    b_heads = jnp.zeros((_HD,), jnp.float32)
    b_heads = b_heads.at[:2].set(ba).at[2].set(bo[0])

    parts1 = _segsum(x, e4, zeros)
    root1 = _root_term(x, W1_root, b1.reshape(1, _D))
    h1 = _combine(parts1, root1, W1_rel)
    parts2 = _segsum(h1, e4, zeros)
    root2 = _root_term(h1, W2_root, b2.reshape(1, _D))
    task_allocation, task_order = _combine_heads(
        parts2, root2, W2_rel, w_heads, b_heads.reshape(1, _HD))
    return (task_allocation, task_order)
